# Initial kernel scaffold; baseline (speedup 1.0000x reference)
import jax
import jax.numpy as jnp
from jax.experimental import pallas as pl


def kernel(x, latent_vector, edge_index, edge_attr, batch_size, nroi, fc1_W, fc1_b, fc1_g, fc1_bb, fc2_W, fc2_b, fc2_g, fc2_bb, fc3_W, fc3_b, fc3_g, fc3_bb, fc4_W, fc4_b, fc4_g, fc4_bb, c1_W0, c1_W1, c1_b, c1_g, c1_bb, c2_W0, c2_W1, c2_b, c2_g, c2_bb):
    raise NotImplementedError("write your pallas kernel here")


# speedup vs baseline: 6.4119x; 6.4119x over previous
import functools

import jax
import jax.numpy as jnp
from jax import lax
from jax.experimental import pallas as pl
from jax.experimental.pallas import tpu as pltpu
from jax.experimental.pallas import tpu_sc as plsc

N = 10000
E = 320000
NPAD = 10240
NC = 2
NS = 16
CHUNK = 80
ROWS_PER_TILE = NPAD // NS
STAGE_ROWS = 128

f32 = jnp.float32
i32 = jnp.int32

_MESH = plsc.VectorSubcoreMesh(core_axis_name="c", subcore_axis_name="s")
_SC_PARAMS = pltpu.CompilerParams(use_tc_tiling_on_sc=False)


def _bn_lrelu(t, g, b):
    m = jnp.mean(t, axis=0)
    v = jnp.mean((t - m) ** 2, axis=0)
    t = g * (t - m) / jnp.sqrt(v + 1e-5) + b
    return jnp.where(t > 0, t, 0.01 * t)


def _deg_body(row_hbm, w_hbm, out_hbm, idx_v, w_v, dbuf, acc):
    cid = lax.axis_index("c")
    sid = lax.axis_index("s")
    zero16 = jnp.zeros((16,), f32)

    def zb(i, c):
        dbuf[pl.ds(i * 16, 16)] = zero16
        return c

    lax.fori_loop(0, ROWS_PER_TILE // 16, zb, 0)
    pltpu.sync_copy(dbuf, acc.at[pl.ds(sid * ROWS_PER_TILE, ROWS_PER_TILE)])
    plsc.subcore_barrier()

    epw = E // (NC * NS)
    base0 = (cid * NS + sid) * epw

    def chunk(k, c):
        base = base0 + k * CHUNK
        pltpu.sync_copy(row_hbm.at[pl.ds(base, CHUNK)], idx_v)
        pltpu.sync_copy(w_hbm.at[pl.ds(base, CHUNK)], w_v)
        pltpu.sync_copy(w_v, acc.at[idx_v], add=True)
        return c

    lax.fori_loop(0, epw // CHUNK, chunk, 0)
    plsc.subcore_barrier()
    pltpu.sync_copy(acc.at[pl.ds(sid * ROWS_PER_TILE, ROWS_PER_TILE)], dbuf)
    pltpu.sync_copy(
        dbuf, out_hbm.at[pl.ds(cid * NPAD + sid * ROWS_PER_TILE, ROWS_PER_TILE)])


_sc_deg = pl.kernel(
    _deg_body,
    out_type=jax.ShapeDtypeStruct((NC * NPAD,), f32),
    mesh=_MESH,
    scratch_types=[
        pltpu.VMEM((CHUNK,), i32),
        pltpu.VMEM((CHUNK,), f32),
        pltpu.VMEM((ROWS_PER_TILE,), f32),
        pltpu.VMEM_SHARED((NPAD,), f32),
    ],
    compiler_params=_SC_PARAMS,
)


def _mp_body(feature_split, dh, table_hbm, col_hbm, row_hbm, w_hbm, out_hbm,
             colv, roww, wv, gidx, rows, acc, sem):
    cid = lax.axis_index("c")
    sid = lax.axis_index("s")
    nv = dh // 16
    zero16 = jnp.zeros((16,), f32)

    def zrow(r, c):
        for f in range(nv):
            rows[r, pl.ds(f * 16, 16)] = zero16
        return c

    lax.fori_loop(0, CHUNK, zrow, 0)
    for k in range(ROWS_PER_TILE // CHUNK):
        pltpu.sync_copy(
            rows, acc.at[pl.ds(sid * ROWS_PER_TILE + k * CHUNK, CHUNK)])
    plsc.subcore_barrier()

    if feature_split:
        epw = E // NS
        ebase = sid * epw
        goff = cid * NPAD
    else:
        epw = E // (NC * NS)
        ebase = (cid * NS + sid) * epw
        goff = None

    def chunk(k, c):
        base = ebase + k * CHUNK
        pltpu.sync_copy(col_hbm.at[pl.ds(base, CHUNK)], colv)
        pltpu.sync_copy(row_hbm.at[pl.ds(base, CHUNK)], roww)
        pltpu.sync_copy(w_hbm.at[pl.ds(base, CHUNK)], wv)
        if feature_split:
            for i in range(CHUNK // 16):
                gidx[pl.ds(i * 16, 16)] = colv[pl.ds(i * 16, 16)] + goff
            idxref = gidx
        else:
            idxref = colv
        pltpu.async_copy(table_hbm.at[idxref], rows, sem).wait()

        def scale(g, c2):
            wg = wv[pl.ds(g * 16, 16)]
            for j in range(16):
                we = wg[j]
                e = g * 16 + j
                for f in range(nv):
                    rows[e, pl.ds(f * 16, 16)] = rows[e, pl.ds(f * 16, 16)] * we
            return c2

        lax.fori_loop(0, CHUNK // 16, scale, 0)
        pltpu.sync_copy(rows, acc.at[roww], add=True)
        return c

    lax.fori_loop(0, epw // CHUNK, chunk, 0)
    plsc.subcore_barrier()
    for k in range(ROWS_PER_TILE // CHUNK):
        r0 = sid * ROWS_PER_TILE + k * CHUNK
        pltpu.sync_copy(acc.at[pl.ds(r0, CHUNK)], rows)
        pltpu.sync_copy(rows, out_hbm.at[pl.ds(cid * NPAD + r0, CHUNK)])


def _make_mp(feature_split, dh, table_rows):
    return pl.kernel(
        functools.partial(_mp_body, feature_split, dh),
        out_type=jax.ShapeDtypeStruct((NC * NPAD, dh), f32),
        mesh=_MESH,
        scratch_types=[
            pltpu.VMEM((CHUNK,), i32),
            pltpu.VMEM((CHUNK,), i32),
            pltpu.VMEM((CHUNK,), f32),
            pltpu.VMEM((CHUNK,), i32),
            pltpu.VMEM((CHUNK, dh), f32),
            pltpu.VMEM_SHARED((NPAD, dh), f32),
            pltpu.SemaphoreType.DMA,
        ],
        compiler_params=_SC_PARAMS,
    )


_sc_mp1 = _make_mp(True, 160, NC * NPAD)
_sc_mp2 = _make_mp(False, 128, NPAD)


def _tc_mlp2_body(h_in, W1, b1, g1, bb1, W2, b2, g2, bb2, out):
    h = _bn_lrelu(h_in[...] @ W1[...] + b1[...], g1[...], bb1[...])
    out[...] = _bn_lrelu(h @ W2[...] + b2[...], g2[...], bb2[...])


def _tc_a3a_body(h4, degp, c1_W1, ys_out, dis_out):
    deg = degp[0, :] + degp[1, :]
    dis = jnp.where(deg > 0, lax.rsqrt(deg), 0.0)
    dis_out[0, :] = dis
    disn = dis[:N][:, None]
    ys = disn * (h4[...] @ c1_W1[...])
    ys_out[0:N, :] = ys[:, :160]
    ys_out[NPAD:NPAD + N, :] = ys[:, 160:]


def _tc_a3b_body(h4, c1_W0, c1_b, z1a_out, z1b_out):
    z1 = h4[...] @ c1_W0[...] + c1_b[...]
    z1a_out[...] = z1[:, :160]
    z1b_out[...] = z1[:, 160:]


def _tc_c1_body(zh, txh, dis, g_h, bb_h, h5_out):
    disn = dis[0, :N][:, None]
    h5_out[...] = _bn_lrelu(zh[...] - disn * txh[0:N, :], g_h[...], bb_h[...])


def _tc_c2_body(h5a, h5b, dis, c2_W0, c2_W1, c2_b, ys2_out, z2_out):
    disn = dis[0, :N][:, None]
    ha = h5a[...]
    hb = h5b[...]
    W1 = c2_W1[...]
    W0 = c2_W0[...]
    ys2_out[0:N, :] = disn * (ha @ W1[:160, :] + hb @ W1[160:, :])
    z2_out[...] = ha @ W0[:160, :] + hb @ W0[160:, :] + c2_b[...]


def _tc_d_body(z2, txp2, dis, c2_g, c2_bb, out):
    tx = txp2[0:N, :] + txp2[NPAD:NPAD + N, :]
    disn = dis[0, :N][:, None]
    out[...] = _bn_lrelu(z2[...] - disn * tx, c2_g[...], c2_bb[...])


def kernel(x, latent_vector, edge_index, edge_attr, batch_size, nroi,
           fc1_W, fc1_b, fc1_g, fc1_bb,
           fc2_W, fc2_b, fc2_g, fc2_bb,
           fc3_W, fc3_b, fc3_g, fc3_bb,
           fc4_W, fc4_b, fc4_g, fc4_bb,
           c1_W0, c1_W1, c1_b, c1_g, c1_bb,
           c2_W0, c2_W1, c2_b, c2_g, c2_bb):
    bsz = latent_vector.shape[0]
    nroi_static = x.shape[0] // bsz
    row = edge_index[0]
    col = edge_index[1]
    lat = jnp.repeat(latent_vector, nroi_static, axis=0)
    xcat = jnp.concatenate([x, lat], axis=-1)

    degp = _sc_deg(row, edge_attr).reshape(2, NPAD)

    h2 = pl.pallas_call(
        _tc_mlp2_body,
        out_shape=jax.ShapeDtypeStruct((N, 160), f32),
    )(xcat, fc1_W, fc1_b, fc1_g, fc1_bb, fc2_W, fc2_b, fc2_g, fc2_bb)

    h4 = pl.pallas_call(
        _tc_mlp2_body,
        out_shape=jax.ShapeDtypeStruct((N, 320), f32),
    )(h2, fc3_W, fc3_b, fc3_g, fc3_bb, fc4_W, fc4_b, fc4_g, fc4_bb)

    ys1s, dis = pl.pallas_call(
        _tc_a3a_body,
        out_shape=[
            jax.ShapeDtypeStruct((NC * NPAD, 160), f32),
            jax.ShapeDtypeStruct((1, NPAD), f32),
        ],
    )(h4, degp, c1_W1)

    z1a, z1b = pl.pallas_call(
        _tc_a3b_body,
        out_shape=[
            jax.ShapeDtypeStruct((N, 160), f32),
            jax.ShapeDtypeStruct((N, 160), f32),
        ],
    )(h4, c1_W0, c1_b)

    txp1 = _sc_mp1(ys1s, col, row, edge_attr).reshape(2, NPAD, 160)

    _c1_call = pl.pallas_call(
        _tc_c1_body,
        out_shape=jax.ShapeDtypeStruct((N, 160), f32),
    )
    h5a = _c1_call(z1a, txp1[0], dis, c1_g[:160], c1_bb[:160])
    h5b = _c1_call(z1b, txp1[1], dis, c1_g[160:], c1_bb[160:])

    ys2, z2 = pl.pallas_call(
        _tc_c2_body,
        out_shape=[
            jax.ShapeDtypeStruct((NPAD, 128), f32),
            jax.ShapeDtypeStruct((N, 128), f32),
        ],
    )(h5a, h5b, dis, c2_W0, c2_W1, c2_b)

    txp2 = _sc_mp2(ys2, col, row, edge_attr)

    h = pl.pallas_call(
        _tc_d_body,
        out_shape=jax.ShapeDtypeStruct((N, 128), f32),
    )(z2, txp2, dis, c2_g, c2_bb)

    return jnp.reshape(h, (bsz, nroi_static, 128))


# speedup vs baseline: 8.2086x; 1.2802x over previous
import functools

import jax
import jax.numpy as jnp
from jax import lax
from jax.experimental import pallas as pl
from jax.experimental.pallas import tpu as pltpu
from jax.experimental.pallas import tpu_sc as plsc

N = 10000
E = 320000
NPAD = 10240
NC = 2
NS = 16
CHUNK = 80
ROWS_PER_TILE = NPAD // NS
STAGE_ROWS = 128

f32 = jnp.float32
i32 = jnp.int32

_MESH = plsc.VectorSubcoreMesh(core_axis_name="c", subcore_axis_name="s")
_SC_PARAMS = pltpu.CompilerParams(use_tc_tiling_on_sc=False)


def _bn_lrelu(t, g, b):
    m = jnp.mean(t, axis=0)
    v = jnp.mean((t - m) ** 2, axis=0)
    t = g * (t - m) / jnp.sqrt(v + 1e-5) + b
    return jnp.where(t > 0, t, 0.01 * t)


def _deg_body(row_hbm, w_hbm, out_hbm, idx0, w0, idx1, w1, dbuf, acc, ssem):
    cid = lax.axis_index("c")
    sid = lax.axis_index("s")
    zero16 = jnp.zeros((16,), f32)

    def zb(i, c):
        dbuf[pl.ds(i * 16, 16)] = zero16
        return c

    lax.fori_loop(0, ROWS_PER_TILE // 16, zb, 0)
    pltpu.sync_copy(dbuf, acc.at[pl.ds(sid * ROWS_PER_TILE, ROWS_PER_TILE)])
    plsc.subcore_barrier()

    epw = E // (NC * NS)
    base0 = (cid * NS + sid) * epw
    nchunks = epw // CHUNK
    bufs = ((idx0, w0), (idx1, w1))

    def load_idx(k, b):
        idx, w = bufs[b]
        base = base0 + k * CHUNK
        pltpu.sync_copy(row_hbm.at[pl.ds(base, CHUNK)], idx)
        pltpu.sync_copy(w_hbm.at[pl.ds(base, CHUNK)], w)

    def start_scatter(b):
        idx, w = bufs[b]
        pltpu.async_copy(w, acc.at[idx], ssem, add=True)

    def wait_scatter(b):
        idx, w = bufs[b]
        pltpu.make_async_copy(w, acc.at[idx], ssem).wait()

    load_idx(0, 0)

    def pair(k2, c):
        for b in range(2):
            k = k2 * 2 + b
            start_scatter(b)

            @pl.when(k >= 1)
            def _():
                wait_scatter(1 - b)

            @pl.when(k + 1 < nchunks)
            def _():
                load_idx(k + 1, 1 - b)
        return c

    lax.fori_loop(0, nchunks // 2, pair, 0)
    if nchunks % 2 == 1:
        start_scatter(0)
        wait_scatter(1)
    wait_scatter(0)
    plsc.subcore_barrier()
    pltpu.sync_copy(acc.at[pl.ds(sid * ROWS_PER_TILE, ROWS_PER_TILE)], dbuf)
    pltpu.sync_copy(
        dbuf, out_hbm.at[pl.ds(cid * NPAD + sid * ROWS_PER_TILE, ROWS_PER_TILE)])


_sc_deg = pl.kernel(
    _deg_body,
    out_type=jax.ShapeDtypeStruct((NC * NPAD,), f32),
    mesh=_MESH,
    scratch_types=[
        pltpu.VMEM((CHUNK,), i32),
        pltpu.VMEM((CHUNK,), f32),
        pltpu.VMEM((CHUNK,), i32),
        pltpu.VMEM((CHUNK,), f32),
        pltpu.VMEM((ROWS_PER_TILE,), f32),
        pltpu.VMEM_SHARED((NPAD,), f32),
        pltpu.SemaphoreType.DMA,
    ],
    compiler_params=_SC_PARAMS,
)


def _mp_body(feature_split, dh, table_hbm, col_hbm, row_hbm, w_hbm, out_hbm,
             colv0, roww0, wv0, rows0, colv1, roww1, wv1, rows1, acc,
             gsem, ssem):
    cid = lax.axis_index("c")
    sid = lax.axis_index("s")
    nv = dh // 16
    zero16 = jnp.zeros((16,), f32)

    def zrow(r, c):
        for f in range(nv):
            rows0[r, pl.ds(f * 16, 16)] = zero16
        return c

    lax.fori_loop(0, CHUNK, zrow, 0)
    for k in range(ROWS_PER_TILE // CHUNK):
        pltpu.sync_copy(
            rows0, acc.at[pl.ds(sid * ROWS_PER_TILE + k * CHUNK, CHUNK)])
    plsc.subcore_barrier()

    if feature_split:
        epw = E // NS
        ebase = sid * epw
        goff = cid * NPAD
    else:
        epw = E // (NC * NS)
        ebase = (cid * NS + sid) * epw
        goff = None
    nchunks = epw // CHUNK
    bufs = ((colv0, roww0, wv0, rows0), (colv1, roww1, wv1, rows1))

    def load_idx(k, b):
        colv, roww, wv, _ = bufs[b]
        base = ebase + k * CHUNK
        pltpu.sync_copy(col_hbm.at[pl.ds(base, CHUNK)], colv)
        pltpu.sync_copy(row_hbm.at[pl.ds(base, CHUNK)], roww)
        pltpu.sync_copy(w_hbm.at[pl.ds(base, CHUNK)], wv)
        if feature_split:
            for i in range(CHUNK // 16):
                colv[pl.ds(i * 16, 16)] = colv[pl.ds(i * 16, 16)] + goff

    def start_gather(b):
        colv, _, _, rows = bufs[b]
        pltpu.async_copy(table_hbm.at[colv], rows, gsem)

    def wait_gather(b):
        colv, _, _, rows = bufs[b]
        pltpu.make_async_copy(table_hbm.at[colv], rows, gsem).wait()

    def start_scatter(b):
        _, roww, _, rows = bufs[b]
        pltpu.async_copy(rows, acc.at[roww], ssem, add=True)

    def wait_scatter(b):
        _, roww, _, rows = bufs[b]
        pltpu.make_async_copy(rows, acc.at[roww], ssem).wait()

    def scale(b):
        _, _, wv, rows = bufs[b]

        def grp(g, c):
            wg = wv[pl.ds(g * 16, 16)]
            for j in range(16):
                we = wg[j]
                e = g * 16 + j
                for f in range(nv):
                    rows[e, pl.ds(f * 16, 16)] = rows[e, pl.ds(f * 16, 16)] * we
            return c

        lax.fori_loop(0, CHUNK // 16, grp, 0)

    load_idx(0, 0)
    start_gather(0)

    def pair(k2, c):
        for b in range(2):
            k = k2 * 2 + b
            wait_gather(b)

            @pl.when(jnp.logical_and(k >= 1, k + 1 < nchunks))
            def _():
                wait_scatter(1 - b)

            @pl.when(k + 1 < nchunks)
            def _():
                load_idx(k + 1, 1 - b)
                start_gather(1 - b)

            scale(b)
            start_scatter(b)
        return c

    lax.fori_loop(0, nchunks // 2, pair, 0)
    if nchunks % 2 == 1:
        wait_gather(0)
        scale(0)
        start_scatter(0)
    wait_scatter(0)
    wait_scatter(1)

    plsc.subcore_barrier()
    for k in range(ROWS_PER_TILE // CHUNK):
        r0 = sid * ROWS_PER_TILE + k * CHUNK
        pltpu.sync_copy(acc.at[pl.ds(r0, CHUNK)], rows0)
        pltpu.sync_copy(rows0, out_hbm.at[pl.ds(cid * NPAD + r0, CHUNK)])


def _make_mp(feature_split, dh, table_rows):
    return pl.kernel(
        functools.partial(_mp_body, feature_split, dh),
        out_type=jax.ShapeDtypeStruct((NC * NPAD, dh), f32),
        mesh=_MESH,
        scratch_types=[
            pltpu.VMEM((CHUNK,), i32),
            pltpu.VMEM((CHUNK,), i32),
            pltpu.VMEM((CHUNK,), f32),
            pltpu.VMEM((CHUNK, dh), f32),
            pltpu.VMEM((CHUNK,), i32),
            pltpu.VMEM((CHUNK,), i32),
            pltpu.VMEM((CHUNK,), f32),
            pltpu.VMEM((CHUNK, dh), f32),
            pltpu.VMEM_SHARED((NPAD, dh), f32),
            pltpu.SemaphoreType.DMA,
            pltpu.SemaphoreType.DMA,
        ],
        compiler_params=_SC_PARAMS,
    )


_sc_mp1 = _make_mp(True, 160, NC * NPAD)
_sc_mp2 = _make_mp(False, 128, NPAD)


def _tc_mlp2_body(h_in, W1, b1, g1, bb1, W2, b2, g2, bb2, out):
    h = _bn_lrelu(h_in[...] @ W1[...] + b1[...], g1[...], bb1[...])
    out[...] = _bn_lrelu(h @ W2[...] + b2[...], g2[...], bb2[...])


def _tc_a3a_body(h4, degp, c1_W1, ys_out, dis_out):
    deg = degp[0, :] + degp[1, :]
    dis = jnp.where(deg > 0, lax.rsqrt(deg), 0.0)
    dis_out[0, :] = dis
    disn = dis[:N][:, None]
    ys = disn * (h4[...] @ c1_W1[...])
    ys_out[0:N, :] = ys[:, :160]
    ys_out[NPAD:NPAD + N, :] = ys[:, 160:]


def _tc_a3b_body(h4, c1_W0, c1_b, z1a_out, z1b_out):
    z1 = h4[...] @ c1_W0[...] + c1_b[...]
    z1a_out[...] = z1[:, :160]
    z1b_out[...] = z1[:, 160:]


def _tc_c1_body(zh, txh, dis, g_h, bb_h, h5_out):
    disn = dis[0, :N][:, None]
    h5_out[...] = _bn_lrelu(zh[...] - disn * txh[0:N, :], g_h[...], bb_h[...])


def _tc_c2_body(h5a, h5b, dis, c2_W0, c2_W1, c2_b, ys2_out, z2_out):
    disn = dis[0, :N][:, None]
    ha = h5a[...]
    hb = h5b[...]
    W1 = c2_W1[...]
    W0 = c2_W0[...]
    ys2_out[0:N, :] = disn * (ha @ W1[:160, :] + hb @ W1[160:, :])
    z2_out[...] = ha @ W0[:160, :] + hb @ W0[160:, :] + c2_b[...]


def _tc_d_body(z2, txp2, dis, c2_g, c2_bb, out):
    tx = txp2[0:N, :] + txp2[NPAD:NPAD + N, :]
    disn = dis[0, :N][:, None]
    out[...] = _bn_lrelu(z2[...] - disn * tx, c2_g[...], c2_bb[...])


def kernel(x, latent_vector, edge_index, edge_attr, batch_size, nroi,
           fc1_W, fc1_b, fc1_g, fc1_bb,
           fc2_W, fc2_b, fc2_g, fc2_bb,
           fc3_W, fc3_b, fc3_g, fc3_bb,
           fc4_W, fc4_b, fc4_g, fc4_bb,
           c1_W0, c1_W1, c1_b, c1_g, c1_bb,
           c2_W0, c2_W1, c2_b, c2_g, c2_bb):
    bsz = latent_vector.shape[0]
    nroi_static = x.shape[0] // bsz
    row = edge_index[0]
    col = edge_index[1]
    lat = jnp.repeat(latent_vector, nroi_static, axis=0)
    xcat = jnp.concatenate([x, lat], axis=-1)

    degp = _sc_deg(row, edge_attr).reshape(2, NPAD)

    h2 = pl.pallas_call(
        _tc_mlp2_body,
        out_shape=jax.ShapeDtypeStruct((N, 160), f32),
    )(xcat, fc1_W, fc1_b, fc1_g, fc1_bb, fc2_W, fc2_b, fc2_g, fc2_bb)

    h4 = pl.pallas_call(
        _tc_mlp2_body,
        out_shape=jax.ShapeDtypeStruct((N, 320), f32),
    )(h2, fc3_W, fc3_b, fc3_g, fc3_bb, fc4_W, fc4_b, fc4_g, fc4_bb)

    ys1s, dis = pl.pallas_call(
        _tc_a3a_body,
        out_shape=[
            jax.ShapeDtypeStruct((NC * NPAD, 160), f32),
            jax.ShapeDtypeStruct((1, NPAD), f32),
        ],
    )(h4, degp, c1_W1)

    z1a, z1b = pl.pallas_call(
        _tc_a3b_body,
        out_shape=[
            jax.ShapeDtypeStruct((N, 160), f32),
            jax.ShapeDtypeStruct((N, 160), f32),
        ],
    )(h4, c1_W0, c1_b)

    txp1 = _sc_mp1(ys1s, col, row, edge_attr).reshape(2, NPAD, 160)

    _c1_call = pl.pallas_call(
        _tc_c1_body,
        out_shape=jax.ShapeDtypeStruct((N, 160), f32),
    )
    h5a = _c1_call(z1a, txp1[0], dis, c1_g[:160], c1_bb[:160])
    h5b = _c1_call(z1b, txp1[1], dis, c1_g[160:], c1_bb[160:])

    ys2, z2 = pl.pallas_call(
        _tc_c2_body,
        out_shape=[
            jax.ShapeDtypeStruct((NPAD, 128), f32),
            jax.ShapeDtypeStruct((N, 128), f32),
        ],
    )(h5a, h5b, dis, c2_W0, c2_W1, c2_b)

    txp2 = _sc_mp2(ys2, col, row, edge_attr)

    h = pl.pallas_call(
        _tc_d_body,
        out_shape=jax.ShapeDtypeStruct((N, 128), f32),
    )(z2, txp2, dis, c2_g, c2_bb)

    return jnp.reshape(h, (bsz, nroi_static, 128))


# speedup vs baseline: 10.9104x; 1.3291x over previous
import functools

import jax
import jax.numpy as jnp
from jax import lax
from jax.experimental import pallas as pl
from jax.experimental.pallas import tpu as pltpu
from jax.experimental.pallas import tpu_sc as plsc

N = 10000
E = 320000
NPAD = 10240
NC = 2
NS = 16
CHUNK = 80
ROWS_PER_TILE = NPAD // NS
STAGE_ROWS = 128

f32 = jnp.float32
i32 = jnp.int32

_MESH = plsc.VectorSubcoreMesh(core_axis_name="c", subcore_axis_name="s")
_SC_PARAMS = pltpu.CompilerParams(use_tc_tiling_on_sc=False)


def _bn_lrelu(t, g, b):
    m = jnp.mean(t, axis=0)
    v = jnp.mean((t - m) ** 2, axis=0)
    t = g * (t - m) / jnp.sqrt(v + 1e-5) + b
    return jnp.where(t > 0, t, 0.01 * t)


def _deg_body(ebl_hbm, out_hbm, ib0, wb0, ib1, wb1, dbuf, acc, ssem):
    cid = lax.axis_index("c")
    sid = lax.axis_index("s")
    zero16 = jnp.zeros((16,), f32)

    def zb(i, c):
        dbuf[pl.ds(i * 16, 16)] = zero16
        return c

    lax.fori_loop(0, ROWS_PER_TILE // 16, zb, 0)
    pltpu.sync_copy(dbuf, acc.at[pl.ds(sid * ROWS_PER_TILE, ROWS_PER_TILE)])
    plsc.subcore_barrier()

    epw = E // (NC * NS)
    nchunks = epw // CHUNK
    cb0 = (cid * NS + sid) * nchunks
    bufs = ((ib0, wb0), (ib1, wb1))

    def load_idx(k, b):
        ib, wb = bufs[b]
        pltpu.sync_copy(ebl_hbm.at[cb0 + k], ib)
        for i in range(CHUNK // 16):
            wb[pl.ds(i * 16, 16)] = lax.bitcast_convert_type(
                ib[2, pl.ds(i * 16, 16)], f32)

    def start_scatter(b):
        ib, wb = bufs[b]
        pltpu.async_copy(wb, acc.at[ib.at[1]], ssem, add=True)

    def wait_scatter(b):
        ib, wb = bufs[b]
        pltpu.make_async_copy(wb, acc.at[ib.at[1]], ssem).wait()

    load_idx(0, 0)

    def pair(k2, c):
        for b in range(2):
            k = k2 * 2 + b
            start_scatter(b)

            @pl.when(k >= 1)
            def _():
                wait_scatter(1 - b)

            @pl.when(k + 1 < nchunks)
            def _():
                load_idx(k + 1, 1 - b)
        return c

    lax.fori_loop(0, nchunks // 2, pair, 0)
    if nchunks % 2 == 1:
        start_scatter(0)
        wait_scatter(1)
    wait_scatter(0)
    plsc.subcore_barrier()
    pltpu.sync_copy(acc.at[pl.ds(sid * ROWS_PER_TILE, ROWS_PER_TILE)], dbuf)
    pltpu.sync_copy(
        dbuf, out_hbm.at[pl.ds(cid * NPAD + sid * ROWS_PER_TILE, ROWS_PER_TILE)])


_sc_deg = pl.kernel(
    _deg_body,
    out_type=jax.ShapeDtypeStruct((NC * NPAD,), f32),
    mesh=_MESH,
    scratch_types=[
        pltpu.VMEM((3, CHUNK), i32),
        pltpu.VMEM((CHUNK,), f32),
        pltpu.VMEM((3, CHUNK), i32),
        pltpu.VMEM((CHUNK,), f32),
        pltpu.VMEM((ROWS_PER_TILE,), f32),
        pltpu.VMEM_SHARED((NPAD,), f32),
        pltpu.SemaphoreType.DMA,
    ],
    compiler_params=_SC_PARAMS,
)


def _mp_body(feature_split, dh, table_hbm, ebl_hbm, out_hbm,
             ib0, rows0, ib1, rows1, acc, gsem, ssem):
    cid = lax.axis_index("c")
    sid = lax.axis_index("s")
    nv = dh // 16
    zero16 = jnp.zeros((16,), f32)

    def zrow(r, c):
        for f in range(nv):
            rows0[r, pl.ds(f * 16, 16)] = zero16
        return c

    lax.fori_loop(0, CHUNK, zrow, 0)
    for k in range(ROWS_PER_TILE // CHUNK):
        pltpu.sync_copy(
            rows0, acc.at[pl.ds(sid * ROWS_PER_TILE + k * CHUNK, CHUNK)])
    plsc.subcore_barrier()

    if feature_split:
        nchunks = E // NS // CHUNK
        cb0 = sid * nchunks
        goff = cid * NPAD
    else:
        nchunks = E // (NC * NS) // CHUNK
        cb0 = (cid * NS + sid) * nchunks
        goff = None
    bufs = ((ib0, rows0), (ib1, rows1))

    def load_idx(k, b):
        ib, _ = bufs[b]
        pltpu.sync_copy(ebl_hbm.at[cb0 + k], ib)
        if feature_split:
            for i in range(CHUNK // 16):
                ib[0, pl.ds(i * 16, 16)] = ib[0, pl.ds(i * 16, 16)] + goff

    def start_gather(b):
        ib, rows = bufs[b]
        pltpu.async_copy(table_hbm.at[ib.at[0]], rows, gsem)

    def wait_gather(b):
        ib, rows = bufs[b]
        pltpu.make_async_copy(table_hbm.at[ib.at[0]], rows, gsem).wait()

    def start_scatter(b):
        ib, rows = bufs[b]
        pltpu.async_copy(rows, acc.at[ib.at[1]], ssem, add=True)

    def wait_scatter(b):
        ib, rows = bufs[b]
        pltpu.make_async_copy(rows, acc.at[ib.at[1]], ssem).wait()

    def scale(b):
        ib, rows = bufs[b]

        def grp(g, c):
            wg = lax.bitcast_convert_type(ib[2, pl.ds(g * 16, 16)], f32)
            for j in range(16):
                we = wg[j]
                e = g * 16 + j
                for f in range(nv):
                    rows[e, pl.ds(f * 16, 16)] = rows[e, pl.ds(f * 16, 16)] * we
            return c

        lax.fori_loop(0, CHUNK // 16, grp, 0)

    load_idx(0, 0)
    start_gather(0)

    def pair(k2, c):
        for b in range(2):
            k = k2 * 2 + b
            wait_gather(b)

            @pl.when(jnp.logical_and(k >= 1, k + 1 < nchunks))
            def _():
                wait_scatter(1 - b)

            @pl.when(k + 1 < nchunks)
            def _():
                load_idx(k + 1, 1 - b)
                start_gather(1 - b)

            scale(b)
            start_scatter(b)
        return c

    lax.fori_loop(0, nchunks // 2, pair, 0)
    if nchunks % 2 == 1:
        wait_gather(0)
        scale(0)
        start_scatter(0)
    wait_scatter(0)
    wait_scatter(1)

    plsc.subcore_barrier()
    for k in range(ROWS_PER_TILE // CHUNK):
        r0 = sid * ROWS_PER_TILE + k * CHUNK
        pltpu.sync_copy(acc.at[pl.ds(r0, CHUNK)], rows0)
        pltpu.sync_copy(rows0, out_hbm.at[pl.ds(cid * NPAD + r0, CHUNK)])


def _make_mp(feature_split, dh, table_rows):
    return pl.kernel(
        functools.partial(_mp_body, feature_split, dh),
        out_type=jax.ShapeDtypeStruct((NC * NPAD, dh), f32),
        mesh=_MESH,
        scratch_types=[
            pltpu.VMEM((3, CHUNK), i32),
            pltpu.VMEM((CHUNK, dh), f32),
            pltpu.VMEM((3, CHUNK), i32),
            pltpu.VMEM((CHUNK, dh), f32),
            pltpu.VMEM_SHARED((NPAD, dh), f32),
            pltpu.SemaphoreType.DMA,
            pltpu.SemaphoreType.DMA,
        ],
        compiler_params=_SC_PARAMS,
    )


_sc_mp1 = _make_mp(True, 160, NC * NPAD)
_sc_mp2 = _make_mp(False, 128, NPAD)


def _tc_mlp2_body(h_in, W1, b1, g1, bb1, W2, b2, g2, bb2, out):
    h = _bn_lrelu(h_in[...] @ W1[...] + b1[...], g1[...], bb1[...])
    out[...] = _bn_lrelu(h @ W2[...] + b2[...], g2[...], bb2[...])


def _tc_a3a_body(h4, degp, c1_W1, ys_out, dis_out):
    deg = degp[0, :] + degp[1, :]
    dis = jnp.where(deg > 0, lax.rsqrt(deg), 0.0)
    dis_out[0, :] = dis
    disn = dis[:N][:, None]
    ys = disn * (h4[...] @ c1_W1[...])
    ys_out[0:N, :] = ys[:, :160]
    ys_out[NPAD:NPAD + N, :] = ys[:, 160:]


def _tc_a3b_body(h4, c1_W0, c1_b, z1a_out, z1b_out):
    z1 = h4[...] @ c1_W0[...] + c1_b[...]
    z1a_out[...] = z1[:, :160]
    z1b_out[...] = z1[:, 160:]


def _tc_c1_body(zh, txh, dis, g_h, bb_h, h5_out):
    disn = dis[0, :N][:, None]
    h5_out[...] = _bn_lrelu(zh[...] - disn * txh[0:N, :], g_h[...], bb_h[...])


def _tc_c2_body(h5a, h5b, dis, c2_W0, c2_W1, c2_b, ys2_out, z2_out):
    disn = dis[0, :N][:, None]
    ha = h5a[...]
    hb = h5b[...]
    W1 = c2_W1[...]
    W0 = c2_W0[...]
    ys2_out[0:N, :] = disn * (ha @ W1[:160, :] + hb @ W1[160:, :])
    z2_out[...] = ha @ W0[:160, :] + hb @ W0[160:, :] + c2_b[...]


def _tc_d_body(z2, txp2, dis, c2_g, c2_bb, out):
    tx = txp2[0:N, :] + txp2[NPAD:NPAD + N, :]
    disn = dis[0, :N][:, None]
    out[...] = _bn_lrelu(z2[...] - disn * tx, c2_g[...], c2_bb[...])


def kernel(x, latent_vector, edge_index, edge_attr, batch_size, nroi,
           fc1_W, fc1_b, fc1_g, fc1_bb,
           fc2_W, fc2_b, fc2_g, fc2_bb,
           fc3_W, fc3_b, fc3_g, fc3_bb,
           fc4_W, fc4_b, fc4_g, fc4_bb,
           c1_W0, c1_W1, c1_b, c1_g, c1_bb,
           c2_W0, c2_W1, c2_b, c2_g, c2_bb):
    bsz = latent_vector.shape[0]
    nroi_static = x.shape[0] // bsz
    row = edge_index[0]
    col = edge_index[1]
    lat = jnp.repeat(latent_vector, nroi_static, axis=0)
    xcat = jnp.concatenate([x, lat], axis=-1)
    ebl = jnp.stack(
        [col.reshape(-1, CHUNK), row.reshape(-1, CHUNK),
         lax.bitcast_convert_type(edge_attr, i32).reshape(-1, CHUNK)], axis=1)

    degp = _sc_deg(ebl).reshape(2, NPAD)

    h2 = pl.pallas_call(
        _tc_mlp2_body,
        out_shape=jax.ShapeDtypeStruct((N, 160), f32),
    )(xcat, fc1_W, fc1_b, fc1_g, fc1_bb, fc2_W, fc2_b, fc2_g, fc2_bb)

    h4 = pl.pallas_call(
        _tc_mlp2_body,
        out_shape=jax.ShapeDtypeStruct((N, 320), f32),
    )(h2, fc3_W, fc3_b, fc3_g, fc3_bb, fc4_W, fc4_b, fc4_g, fc4_bb)

    ys1s, dis = pl.pallas_call(
        _tc_a3a_body,
        out_shape=[
            jax.ShapeDtypeStruct((NC * NPAD, 160), f32),
            jax.ShapeDtypeStruct((1, NPAD), f32),
        ],
    )(h4, degp, c1_W1)

    z1a, z1b = pl.pallas_call(
        _tc_a3b_body,
        out_shape=[
            jax.ShapeDtypeStruct((N, 160), f32),
            jax.ShapeDtypeStruct((N, 160), f32),
        ],
    )(h4, c1_W0, c1_b)

    txp1 = _sc_mp1(ys1s, ebl).reshape(2, NPAD, 160)

    _c1_call = pl.pallas_call(
        _tc_c1_body,
        out_shape=jax.ShapeDtypeStruct((N, 160), f32),
    )
    h5a = _c1_call(z1a, txp1[0], dis, c1_g[:160], c1_bb[:160])
    h5b = _c1_call(z1b, txp1[1], dis, c1_g[160:], c1_bb[160:])

    ys2, z2 = pl.pallas_call(
        _tc_c2_body,
        out_shape=[
            jax.ShapeDtypeStruct((NPAD, 128), f32),
            jax.ShapeDtypeStruct((N, 128), f32),
        ],
    )(h5a, h5b, dis, c2_W0, c2_W1, c2_b)

    txp2 = _sc_mp2(ys2, ebl)

    h = pl.pallas_call(
        _tc_d_body,
        out_shape=jax.ShapeDtypeStruct((N, 128), f32),
    )(z2, txp2, dis, c2_g, c2_bb)

    return jnp.reshape(h, (bsz, nroi_static, 128))


# speedup vs baseline: 13.1743x; 1.2075x over previous
import functools

import jax
import jax.numpy as jnp
from jax import lax
from jax.experimental import pallas as pl
from jax.experimental.pallas import tpu as pltpu
from jax.experimental.pallas import tpu_sc as plsc

N = 10000
E = 320000
NPAD = 10240
NC = 2
NS = 16
CHUNK = 80
ROWS_PER_TILE = NPAD // NS
STAGE_ROWS = 128

f32 = jnp.float32
i32 = jnp.int32

_MESH = plsc.VectorSubcoreMesh(core_axis_name="c", subcore_axis_name="s")
_SC_PARAMS = pltpu.CompilerParams(use_tc_tiling_on_sc=False)


def _bn_lrelu(t, g, b):
    m = jnp.mean(t, axis=0)
    v = jnp.mean((t - m) ** 2, axis=0)
    t = g * (t - m) / jnp.sqrt(v + 1e-5) + b
    return jnp.where(t > 0, t, 0.01 * t)


def _deg_body(ebl_hbm, out_hbm, ib0, wb0, ib1, wb1, dbuf, acc, ssem):
    cid = lax.axis_index("c")
    sid = lax.axis_index("s")
    zero16 = jnp.zeros((16,), f32)

    def zb(i, c):
        dbuf[pl.ds(i * 16, 16)] = zero16
        return c

    lax.fori_loop(0, ROWS_PER_TILE // 16, zb, 0)
    pltpu.sync_copy(dbuf, acc.at[pl.ds(sid * ROWS_PER_TILE, ROWS_PER_TILE)])
    plsc.subcore_barrier()

    epw = E // (NC * NS)
    nchunks = epw // CHUNK
    cb0 = (cid * NS + sid) * nchunks
    bufs = ((ib0, wb0), (ib1, wb1))

    def load_idx(k, b):
        ib, wb = bufs[b]
        pltpu.sync_copy(ebl_hbm.at[cb0 + k], ib)
        for i in range(CHUNK // 16):
            wb[pl.ds(i * 16, 16)] = lax.bitcast_convert_type(
                ib[2, pl.ds(i * 16, 16)], f32)

    def start_scatter(b):
        ib, wb = bufs[b]
        pltpu.async_copy(wb, acc.at[ib.at[1]], ssem, add=True)

    def wait_scatter(b):
        ib, wb = bufs[b]
        pltpu.make_async_copy(wb, acc.at[ib.at[1]], ssem).wait()

    load_idx(0, 0)

    def pair(k2, c):
        for b in range(2):
            k = k2 * 2 + b
            start_scatter(b)

            @pl.when(k >= 1)
            def _():
                wait_scatter(1 - b)

            @pl.when(k + 1 < nchunks)
            def _():
                load_idx(k + 1, 1 - b)
        return c

    lax.fori_loop(0, nchunks // 2, pair, 0)
    if nchunks % 2 == 1:
        start_scatter(0)
        wait_scatter(1)
    wait_scatter(0)
    plsc.subcore_barrier()
    pltpu.sync_copy(acc.at[pl.ds(sid * ROWS_PER_TILE, ROWS_PER_TILE)], dbuf)
    pltpu.sync_copy(
        dbuf, out_hbm.at[pl.ds(cid * NPAD + sid * ROWS_PER_TILE, ROWS_PER_TILE)])


_sc_deg = pl.kernel(
    _deg_body,
    out_type=jax.ShapeDtypeStruct((NC * NPAD,), f32),
    mesh=_MESH,
    scratch_types=[
        pltpu.VMEM((3, CHUNK), i32),
        pltpu.VMEM((CHUNK,), f32),
        pltpu.VMEM((3, CHUNK), i32),
        pltpu.VMEM((CHUNK,), f32),
        pltpu.VMEM((ROWS_PER_TILE,), f32),
        pltpu.VMEM_SHARED((NPAD,), f32),
        pltpu.SemaphoreType.DMA,
    ],
    compiler_params=_SC_PARAMS,
)


def _mp_body(feature_split, dh, table_hbm, ebl_hbm, out_hbm,
             ib0, ib1, ib2, ib3, rows0, rows1, acc,
             gsem, ssem0, ssem1, isem0, isem1, isem2, isem3):
    cid = lax.axis_index("c")
    sid = lax.axis_index("s")
    nv = dh // 16
    zero16 = jnp.zeros((16,), f32)

    if feature_split:
        nchunks = E // NS // CHUNK
        cb0 = sid * nchunks
        goff = cid * NPAD
    else:
        nchunks = E // (NC * NS) // CHUNK
        cb0 = (cid * NS + sid) * nchunks
        goff = None
    ibs = (ib0, ib1, ib2, ib3)
    isems = (isem0, isem1, isem2, isem3)
    rowss = (rows0, rows1)
    ssems = (ssem0, ssem1)

    def start_idx(k, q):
        pltpu.async_copy(ebl_hbm.at[cb0 + k], ibs[q], isems[q])

    def wait_idx(q):
        pltpu.make_async_copy(ebl_hbm.at[cb0], ibs[q], isems[q]).wait()
        if feature_split:
            for i in range(CHUNK // 16):
                ibs[q][0, pl.ds(i * 16, 16)] = (
                    ibs[q][0, pl.ds(i * 16, 16)] + goff)

    def start_gather(q, r):
        pltpu.async_copy(table_hbm.at[ibs[q].at[0]], rowss[r], gsem)

    def wait_gather(q, r):
        pltpu.make_async_copy(table_hbm.at[ibs[q].at[0]], rowss[r], gsem).wait()

    def start_scatter(q, r):
        pltpu.async_copy(rowss[r], acc.at[ibs[q].at[1]], ssems[r], add=True)

    def wait_scatter(q, r):
        pltpu.make_async_copy(rowss[r], acc.at[ibs[q].at[1]], ssems[r]).wait()

    def scale(q, r):
        ib = ibs[q]
        rows = rowss[r]

        def grp(g, c):
            wg = lax.bitcast_convert_type(ib[2, pl.ds(g * 16, 16)], f32)
            for j in range(16):
                we = wg[j]
                e = g * 16 + j
                for f in range(nv):
                    rows[e, pl.ds(f * 16, 16)] = rows[e, pl.ds(f * 16, 16)] * we
            return c

        lax.fori_loop(0, CHUNK // 16, grp, 0)

    start_idx(0, 0)
    start_idx(1, 1)
    start_idx(2, 2)

    def zrow(r, c):
        for f in range(nv):
            rows0[r, pl.ds(f * 16, 16)] = zero16
        return c

    lax.fori_loop(0, CHUNK, zrow, 0)
    for k in range(ROWS_PER_TILE // CHUNK):
        pltpu.sync_copy(
            rows0, acc.at[pl.ds(sid * ROWS_PER_TILE + k * CHUNK, CHUNK)])
    plsc.subcore_barrier()

    wait_idx(0)
    start_gather(0, 0)

    def chunk_step(k, j):
        q, r = j, j % 2
        wait_gather(q, r)

        @pl.when(k >= 1)
        def _():
            wait_scatter((j + 3) % 4, 1 - r)

        wait_idx((j + 1) % 4)
        start_gather((j + 1) % 4, 1 - r)

        @pl.when(k + 3 < nchunks)
        def _():
            start_idx(k + 3, (j + 3) % 4)

        scale(q, r)
        start_scatter(q, r)

    nmain = (nchunks - 1) // 4 * 4

    def quad(k4, c):
        for j in range(4):
            chunk_step(k4 * 4 + j, j)
        return c

    lax.fori_loop(0, nmain // 4, quad, 0)
    for k in range(nmain, nchunks):
        j = k % 4
        q, r = j, j % 2
        wait_gather(q, r)
        wait_scatter((j + 3) % 4, 1 - r)
        if k + 1 < nchunks:
            wait_idx((j + 1) % 4)
            start_gather((j + 1) % 4, 1 - r)
        scale(q, r)
        start_scatter(q, r)
    wait_scatter((nchunks - 1) % 4, (nchunks - 1) % 2)

    plsc.subcore_barrier()
    for k in range(ROWS_PER_TILE // CHUNK):
        r0 = sid * ROWS_PER_TILE + k * CHUNK
        pltpu.sync_copy(acc.at[pl.ds(r0, CHUNK)], rows0)
        pltpu.sync_copy(rows0, out_hbm.at[pl.ds(cid * NPAD + r0, CHUNK)])


def _make_mp(feature_split, dh, table_rows):
    return pl.kernel(
        functools.partial(_mp_body, feature_split, dh),
        out_type=jax.ShapeDtypeStruct((NC * NPAD, dh), f32),
        mesh=_MESH,
        scratch_types=[
            pltpu.VMEM((3, CHUNK), i32),
            pltpu.VMEM((3, CHUNK), i32),
            pltpu.VMEM((3, CHUNK), i32),
            pltpu.VMEM((3, CHUNK), i32),
            pltpu.VMEM((CHUNK, dh), f32),
            pltpu.VMEM((CHUNK, dh), f32),
            pltpu.VMEM_SHARED((NPAD, dh), f32),
            pltpu.SemaphoreType.DMA,
            pltpu.SemaphoreType.DMA,
            pltpu.SemaphoreType.DMA,
            pltpu.SemaphoreType.DMA,
            pltpu.SemaphoreType.DMA,
            pltpu.SemaphoreType.DMA,
            pltpu.SemaphoreType.DMA,
        ],
        compiler_params=_SC_PARAMS,
    )


_sc_mp1 = _make_mp(True, 160, NC * NPAD)
_sc_mp2 = _make_mp(False, 128, NPAD)


def _tc_mlp2_body(h_in, W1, b1, g1, bb1, W2, b2, g2, bb2, out):
    h = _bn_lrelu(h_in[...] @ W1[...] + b1[...], g1[...], bb1[...])
    out[...] = _bn_lrelu(h @ W2[...] + b2[...], g2[...], bb2[...])


def _tc_a3a_body(h4, degp, c1_W1, ys_out, dis_out):
    deg = degp[0, :] + degp[1, :]
    dis = jnp.where(deg > 0, lax.rsqrt(deg), 0.0)
    dis_out[0, :] = dis
    disn = dis[:N][:, None]
    ys = disn * (h4[...] @ c1_W1[...])
    ys_out[0:N, :] = ys[:, :160]
    ys_out[NPAD:NPAD + N, :] = ys[:, 160:]


def _tc_a3b_body(h4, c1_W0, c1_b, z1a_out, z1b_out):
    z1 = h4[...] @ c1_W0[...] + c1_b[...]
    z1a_out[...] = z1[:, :160]
    z1b_out[...] = z1[:, 160:]


def _tc_c1_body(zh, txh, dis, g_h, bb_h, h5_out):
    disn = dis[0, :N][:, None]
    h5_out[...] = _bn_lrelu(zh[...] - disn * txh[0:N, :], g_h[...], bb_h[...])


def _tc_c2_body(h5a, h5b, dis, c2_W0, c2_W1, c2_b, ys2_out, z2_out):
    disn = dis[0, :N][:, None]
    ha = h5a[...]
    hb = h5b[...]
    W1 = c2_W1[...]
    W0 = c2_W0[...]
    ys2_out[0:N, :] = disn * (ha @ W1[:160, :] + hb @ W1[160:, :])
    z2_out[...] = ha @ W0[:160, :] + hb @ W0[160:, :] + c2_b[...]


def _tc_d_body(z2, txp2, dis, c2_g, c2_bb, out):
    tx = txp2[0:N, :] + txp2[NPAD:NPAD + N, :]
    disn = dis[0, :N][:, None]
    out[...] = _bn_lrelu(z2[...] - disn * tx, c2_g[...], c2_bb[...])


def kernel(x, latent_vector, edge_index, edge_attr, batch_size, nroi,
           fc1_W, fc1_b, fc1_g, fc1_bb,
           fc2_W, fc2_b, fc2_g, fc2_bb,
           fc3_W, fc3_b, fc3_g, fc3_bb,
           fc4_W, fc4_b, fc4_g, fc4_bb,
           c1_W0, c1_W1, c1_b, c1_g, c1_bb,
           c2_W0, c2_W1, c2_b, c2_g, c2_bb):
    bsz = latent_vector.shape[0]
    nroi_static = x.shape[0] // bsz
    row = edge_index[0]
    col = edge_index[1]
    lat = jnp.repeat(latent_vector, nroi_static, axis=0)
    xcat = jnp.concatenate([x, lat], axis=-1)
    ebl = jnp.stack(
        [col.reshape(-1, CHUNK), row.reshape(-1, CHUNK),
         lax.bitcast_convert_type(edge_attr, i32).reshape(-1, CHUNK)], axis=1)

    degp = _sc_deg(ebl).reshape(2, NPAD)

    h2 = pl.pallas_call(
        _tc_mlp2_body,
        out_shape=jax.ShapeDtypeStruct((N, 160), f32),
    )(xcat, fc1_W, fc1_b, fc1_g, fc1_bb, fc2_W, fc2_b, fc2_g, fc2_bb)

    h4 = pl.pallas_call(
        _tc_mlp2_body,
        out_shape=jax.ShapeDtypeStruct((N, 320), f32),
    )(h2, fc3_W, fc3_b, fc3_g, fc3_bb, fc4_W, fc4_b, fc4_g, fc4_bb)

    ys1s, dis = pl.pallas_call(
        _tc_a3a_body,
        out_shape=[
            jax.ShapeDtypeStruct((NC * NPAD, 160), f32),
            jax.ShapeDtypeStruct((1, NPAD), f32),
        ],
    )(h4, degp, c1_W1)

    z1a, z1b = pl.pallas_call(
        _tc_a3b_body,
        out_shape=[
            jax.ShapeDtypeStruct((N, 160), f32),
            jax.ShapeDtypeStruct((N, 160), f32),
        ],
    )(h4, c1_W0, c1_b)

    txp1 = _sc_mp1(ys1s, ebl).reshape(2, NPAD, 160)

    _c1_call = pl.pallas_call(
        _tc_c1_body,
        out_shape=jax.ShapeDtypeStruct((N, 160), f32),
    )
    h5a = _c1_call(z1a, txp1[0], dis, c1_g[:160], c1_bb[:160])
    h5b = _c1_call(z1b, txp1[1], dis, c1_g[160:], c1_bb[160:])

    ys2, z2 = pl.pallas_call(
        _tc_c2_body,
        out_shape=[
            jax.ShapeDtypeStruct((NPAD, 128), f32),
            jax.ShapeDtypeStruct((N, 128), f32),
        ],
    )(h5a, h5b, dis, c2_W0, c2_W1, c2_b)

    txp2 = _sc_mp2(ys2, ebl)

    h = pl.pallas_call(
        _tc_d_body,
        out_shape=jax.ShapeDtypeStruct((N, 128), f32),
    )(z2, txp2, dis, c2_g, c2_bb)

    return jnp.reshape(h, (bsz, nroi_static, 128))


# speedup vs baseline: 13.3616x; 1.0142x over previous
import functools

import jax
import jax.numpy as jnp
from jax import lax
from jax.experimental import pallas as pl
from jax.experimental.pallas import tpu as pltpu
from jax.experimental.pallas import tpu_sc as plsc

N = 10000
E = 320000
NPAD = 10240
NC = 2
NS = 16
CHUNK = 80
ROWS_PER_TILE = NPAD // NS
STAGE_ROWS = 128

f32 = jnp.float32
i32 = jnp.int32

_MESH = plsc.VectorSubcoreMesh(core_axis_name="c", subcore_axis_name="s")
_SC_PARAMS = pltpu.CompilerParams(use_tc_tiling_on_sc=False)


def _bn_lrelu(t, g, b):
    m = jnp.mean(t, axis=0)
    v = jnp.mean((t - m) ** 2, axis=0)
    t = g * (t - m) / jnp.sqrt(v + 1e-5) + b
    return jnp.where(t > 0, t, 0.01 * t)


def _deg_body(ebl_hbm, out_hbm, ib0, ib1, ib2, ib3, wb0, wb1, dbuf, acc,
              ssem0, ssem1, isem0, isem1, isem2, isem3):
    cid = lax.axis_index("c")
    sid = lax.axis_index("s")
    zero16 = jnp.zeros((16,), f32)

    epw = E // (NC * NS)
    nchunks = epw // CHUNK
    cb0 = (cid * NS + sid) * nchunks
    ibs = (ib0, ib1, ib2, ib3)
    isems = (isem0, isem1, isem2, isem3)
    wbs = (wb0, wb1)
    ssems = (ssem0, ssem1)

    def start_idx(k, q):
        pltpu.async_copy(ebl_hbm.at[cb0 + k], ibs[q], isems[q])

    def wait_idx(q):
        pltpu.make_async_copy(ebl_hbm.at[cb0], ibs[q], isems[q]).wait()

    def start_scatter(q, r):
        pltpu.async_copy(wbs[r], acc.at[ibs[q].at[1]], ssems[r], add=True)

    def wait_scatter(q, r):
        pltpu.make_async_copy(wbs[r], acc.at[ibs[q].at[1]], ssems[r]).wait()

    start_idx(0, 0)
    start_idx(1, 1)

    def zb(i, c):
        dbuf[pl.ds(i * 16, 16)] = zero16
        return c

    lax.fori_loop(0, ROWS_PER_TILE // 16, zb, 0)
    pltpu.sync_copy(dbuf, acc.at[pl.ds(sid * ROWS_PER_TILE, ROWS_PER_TILE)])
    plsc.subcore_barrier()

    def chunk_step(k, j, static_tail):
        q, r = j, j % 2

        def waitprev():
            wait_scatter((j + 2) % 4, r)

        if static_tail:
            waitprev()
        else:
            pl.when(k >= 2)(waitprev)
        wait_idx(q)
        ib = ibs[q]
        wb = wbs[r]
        for i in range(CHUNK // 16):
            wb[pl.ds(i * 16, 16)] = lax.bitcast_convert_type(
                ib[2, pl.ds(i * 16, 16)], f32)
        start_scatter(q, r)
        if static_tail:
            if k + 2 < nchunks:
                start_idx(k + 2, (j + 2) % 4)
        else:
            @pl.when(k + 2 < nchunks)
            def _():
                start_idx(k + 2, (j + 2) % 4)

    nmain = (nchunks - 2) // 4 * 4

    def quad(k4, c):
        for j in range(4):
            chunk_step(k4 * 4 + j, j, False)
        return c

    lax.fori_loop(0, nmain // 4, quad, 0)
    for k in range(nmain, nchunks):
        chunk_step(k, k % 4, True)
    wait_scatter((nchunks - 2) % 4, (nchunks - 2) % 2)
    wait_scatter((nchunks - 1) % 4, (nchunks - 1) % 2)
    plsc.subcore_barrier()
    pltpu.sync_copy(acc.at[pl.ds(sid * ROWS_PER_TILE, ROWS_PER_TILE)], dbuf)
    pltpu.sync_copy(
        dbuf, out_hbm.at[pl.ds(cid * NPAD + sid * ROWS_PER_TILE, ROWS_PER_TILE)])


_sc_deg = pl.kernel(
    _deg_body,
    out_type=jax.ShapeDtypeStruct((NC * NPAD,), f32),
    mesh=_MESH,
    scratch_types=[
        pltpu.VMEM((3, CHUNK), i32),
        pltpu.VMEM((3, CHUNK), i32),
        pltpu.VMEM((3, CHUNK), i32),
        pltpu.VMEM((3, CHUNK), i32),
        pltpu.VMEM((CHUNK,), f32),
        pltpu.VMEM((CHUNK,), f32),
        pltpu.VMEM((ROWS_PER_TILE,), f32),
        pltpu.VMEM_SHARED((NPAD,), f32),
        pltpu.SemaphoreType.DMA,
        pltpu.SemaphoreType.DMA,
        pltpu.SemaphoreType.DMA,
        pltpu.SemaphoreType.DMA,
        pltpu.SemaphoreType.DMA,
        pltpu.SemaphoreType.DMA,
    ],
    compiler_params=_SC_PARAMS,
)


def _mp_body(feature_split, dh, table_hbm, ebl_hbm, out_hbm,
             ib0, ib1, ib2, ib3, rows0, rows1, acc,
             gsem, ssem0, ssem1, isem0, isem1, isem2, isem3):
    cid = lax.axis_index("c")
    sid = lax.axis_index("s")
    nv = dh // 16
    zero16 = jnp.zeros((16,), f32)

    if feature_split:
        nchunks = E // NS // CHUNK
        cb0 = sid * nchunks
        goff = cid * NPAD
    else:
        nchunks = E // (NC * NS) // CHUNK
        cb0 = (cid * NS + sid) * nchunks
        goff = None
    ibs = (ib0, ib1, ib2, ib3)
    isems = (isem0, isem1, isem2, isem3)
    rowss = (rows0, rows1)
    ssems = (ssem0, ssem1)

    def start_idx(k, q):
        pltpu.async_copy(ebl_hbm.at[cb0 + k], ibs[q], isems[q])

    def wait_idx(q):
        pltpu.make_async_copy(ebl_hbm.at[cb0], ibs[q], isems[q]).wait()
        if feature_split:
            for i in range(CHUNK // 16):
                ibs[q][0, pl.ds(i * 16, 16)] = (
                    ibs[q][0, pl.ds(i * 16, 16)] + goff)

    def start_gather(q, r):
        pltpu.async_copy(table_hbm.at[ibs[q].at[0]], rowss[r], gsem)

    def wait_gather(q, r):
        pltpu.make_async_copy(table_hbm.at[ibs[q].at[0]], rowss[r], gsem).wait()

    def start_scatter(q, r):
        pltpu.async_copy(rowss[r], acc.at[ibs[q].at[1]], ssems[r], add=True)

    def wait_scatter(q, r):
        pltpu.make_async_copy(rowss[r], acc.at[ibs[q].at[1]], ssems[r]).wait()

    def scale(q, r):
        ib = ibs[q]
        rows = rowss[r]

        def grp(g, c):
            wg = lax.bitcast_convert_type(ib[2, pl.ds(g * 16, 16)], f32)
            for j in range(16):
                we = wg[j]
                e = g * 16 + j
                for f in range(nv):
                    rows[e, pl.ds(f * 16, 16)] = rows[e, pl.ds(f * 16, 16)] * we
            return c

        lax.fori_loop(0, CHUNK // 16, grp, 0)

    start_idx(0, 0)
    start_idx(1, 1)
    start_idx(2, 2)

    def zrow(r, c):
        for f in range(nv):
            rows0[r, pl.ds(f * 16, 16)] = zero16
        return c

    lax.fori_loop(0, CHUNK, zrow, 0)
    for k in range(ROWS_PER_TILE // CHUNK):
        pltpu.sync_copy(
            rows0, acc.at[pl.ds(sid * ROWS_PER_TILE + k * CHUNK, CHUNK)])
    plsc.subcore_barrier()

    wait_idx(0)
    start_gather(0, 0)

    def chunk_step(k, j):
        q, r = j, j % 2
        wait_gather(q, r)

        @pl.when(k >= 1)
        def _():
            wait_scatter((j + 3) % 4, 1 - r)

        wait_idx((j + 1) % 4)
        start_gather((j + 1) % 4, 1 - r)

        @pl.when(k + 3 < nchunks)
        def _():
            start_idx(k + 3, (j + 3) % 4)

        scale(q, r)
        start_scatter(q, r)

    nmain = (nchunks - 1) // 4 * 4

    def quad(k4, c):
        for j in range(4):
            chunk_step(k4 * 4 + j, j)
        return c

    lax.fori_loop(0, nmain // 4, quad, 0)
    for k in range(nmain, nchunks):
        j = k % 4
        q, r = j, j % 2
        wait_gather(q, r)
        wait_scatter((j + 3) % 4, 1 - r)
        if k + 1 < nchunks:
            wait_idx((j + 1) % 4)
            start_gather((j + 1) % 4, 1 - r)
        scale(q, r)
        start_scatter(q, r)
    wait_scatter((nchunks - 1) % 4, (nchunks - 1) % 2)

    plsc.subcore_barrier()
    for k in range(ROWS_PER_TILE // CHUNK):
        r0 = sid * ROWS_PER_TILE + k * CHUNK
        pltpu.sync_copy(acc.at[pl.ds(r0, CHUNK)], rows0)
        pltpu.sync_copy(rows0, out_hbm.at[pl.ds(cid * NPAD + r0, CHUNK)])


def _make_mp(feature_split, dh, table_rows):
    return pl.kernel(
        functools.partial(_mp_body, feature_split, dh),
        out_type=jax.ShapeDtypeStruct((NC * NPAD, dh), f32),
        mesh=_MESH,
        scratch_types=[
            pltpu.VMEM((3, CHUNK), i32),
            pltpu.VMEM((3, CHUNK), i32),
            pltpu.VMEM((3, CHUNK), i32),
            pltpu.VMEM((3, CHUNK), i32),
            pltpu.VMEM((CHUNK, dh), f32),
            pltpu.VMEM((CHUNK, dh), f32),
            pltpu.VMEM_SHARED((NPAD, dh), f32),
            pltpu.SemaphoreType.DMA,
            pltpu.SemaphoreType.DMA,
            pltpu.SemaphoreType.DMA,
            pltpu.SemaphoreType.DMA,
            pltpu.SemaphoreType.DMA,
            pltpu.SemaphoreType.DMA,
            pltpu.SemaphoreType.DMA,
        ],
        compiler_params=_SC_PARAMS,
    )


_sc_mp1 = _make_mp(True, 160, NC * NPAD)
_sc_mp2 = _make_mp(False, 128, NPAD)


def _tc_mlp2_body(h_in, W1, b1, g1, bb1, W2, b2, g2, bb2, out):
    h = _bn_lrelu(h_in[...] @ W1[...] + b1[...], g1[...], bb1[...])
    out[...] = _bn_lrelu(h @ W2[...] + b2[...], g2[...], bb2[...])


def _tc_a3a_body(h4, degp, c1_W1, ys_out, dis_out):
    deg = degp[0, :] + degp[1, :]
    dis = jnp.where(deg > 0, lax.rsqrt(deg), 0.0)
    dis_out[0, :] = dis
    disn = dis[:N][:, None]
    ys = disn * (h4[...] @ c1_W1[...])
    ys_out[0:N, :] = ys[:, :160]
    ys_out[NPAD:NPAD + N, :] = ys[:, 160:]


def _tc_a3b_body(h4, c1_W0, c1_b, z1a_out, z1b_out):
    z1 = h4[...] @ c1_W0[...] + c1_b[...]
    z1a_out[...] = z1[:, :160]
    z1b_out[...] = z1[:, 160:]


def _tc_c1_body(zh, txh, dis, g_h, bb_h, h5_out):
    disn = dis[0, :N][:, None]
    h5_out[...] = _bn_lrelu(zh[...] - disn * txh[0:N, :], g_h[...], bb_h[...])


def _tc_c2_body(h5a, h5b, dis, c2_W0, c2_W1, c2_b, ys2_out, z2_out):
    disn = dis[0, :N][:, None]
    ha = h5a[...]
    hb = h5b[...]
    W1 = c2_W1[...]
    W0 = c2_W0[...]
    ys2_out[0:N, :] = disn * (ha @ W1[:160, :] + hb @ W1[160:, :])
    z2_out[...] = ha @ W0[:160, :] + hb @ W0[160:, :] + c2_b[...]


def _tc_d_body(z2, txp2, dis, c2_g, c2_bb, out):
    tx = txp2[0:N, :] + txp2[NPAD:NPAD + N, :]
    disn = dis[0, :N][:, None]
    out[...] = _bn_lrelu(z2[...] - disn * tx, c2_g[...], c2_bb[...])


def kernel(x, latent_vector, edge_index, edge_attr, batch_size, nroi,
           fc1_W, fc1_b, fc1_g, fc1_bb,
           fc2_W, fc2_b, fc2_g, fc2_bb,
           fc3_W, fc3_b, fc3_g, fc3_bb,
           fc4_W, fc4_b, fc4_g, fc4_bb,
           c1_W0, c1_W1, c1_b, c1_g, c1_bb,
           c2_W0, c2_W1, c2_b, c2_g, c2_bb):
    bsz = latent_vector.shape[0]
    nroi_static = x.shape[0] // bsz
    row = edge_index[0]
    col = edge_index[1]
    lat = jnp.repeat(latent_vector, nroi_static, axis=0)
    xcat = jnp.concatenate([x, lat], axis=-1)
    ebl = jnp.stack(
        [col.reshape(-1, CHUNK), row.reshape(-1, CHUNK),
         lax.bitcast_convert_type(edge_attr, i32).reshape(-1, CHUNK)], axis=1)

    degp = _sc_deg(ebl).reshape(2, NPAD)

    h2 = pl.pallas_call(
        _tc_mlp2_body,
        out_shape=jax.ShapeDtypeStruct((N, 160), f32),
    )(xcat, fc1_W, fc1_b, fc1_g, fc1_bb, fc2_W, fc2_b, fc2_g, fc2_bb)

    h4 = pl.pallas_call(
        _tc_mlp2_body,
        out_shape=jax.ShapeDtypeStruct((N, 320), f32),
    )(h2, fc3_W, fc3_b, fc3_g, fc3_bb, fc4_W, fc4_b, fc4_g, fc4_bb)

    ys1s, dis = pl.pallas_call(
        _tc_a3a_body,
        out_shape=[
            jax.ShapeDtypeStruct((NC * NPAD, 160), f32),
            jax.ShapeDtypeStruct((1, NPAD), f32),
        ],
    )(h4, degp, c1_W1)

    z1a, z1b = pl.pallas_call(
        _tc_a3b_body,
        out_shape=[
            jax.ShapeDtypeStruct((N, 160), f32),
            jax.ShapeDtypeStruct((N, 160), f32),
        ],
    )(h4, c1_W0, c1_b)

    txp1 = _sc_mp1(ys1s, ebl).reshape(2, NPAD, 160)

    _c1_call = pl.pallas_call(
        _tc_c1_body,
        out_shape=jax.ShapeDtypeStruct((N, 160), f32),
    )
    h5a = _c1_call(z1a, txp1[0], dis, c1_g[:160], c1_bb[:160])
    h5b = _c1_call(z1b, txp1[1], dis, c1_g[160:], c1_bb[160:])

    ys2, z2 = pl.pallas_call(
        _tc_c2_body,
        out_shape=[
            jax.ShapeDtypeStruct((NPAD, 128), f32),
            jax.ShapeDtypeStruct((N, 128), f32),
        ],
    )(h5a, h5b, dis, c2_W0, c2_W1, c2_b)

    txp2 = _sc_mp2(ys2, ebl)

    h = pl.pallas_call(
        _tc_d_body,
        out_shape=jax.ShapeDtypeStruct((N, 128), f32),
    )(z2, txp2, dis, c2_g, c2_bb)

    return jnp.reshape(h, (bsz, nroi_static, 128))
